# Initial kernel scaffold; baseline (speedup 1.0000x reference)
import functools

import jax
import jax.numpy as jnp
from jax import lax
from jax.experimental import pallas as pl
from jax.experimental.pallas import tpu as pltpu, tpu_sc as plsc

N = 10000
E = 320000
D = 128
NPAD = 10240
TRASH = 10000
NW = 32
EW = E // NW
R = 1000
GRID = N // R

_mesh = plsc.VectorSubcoreMesh(core_axis_name="c", subcore_axis_name="s")

_CH = 2000


@functools.partial(
    pl.kernel,
    out_type=[
        jax.ShapeDtypeStruct((E,), jnp.int32),
        jax.ShapeDtypeStruct((2, NPAD), jnp.float32),
    ],
    scratch_types=[
        pltpu.VMEM((_CH,), jnp.int32),
        pltpu.VMEM((_CH,), jnp.int32),
        pltpu.VMEM((_CH,), jnp.int32),
        pltpu.VMEM((NPAD,), jnp.float32),
        pltpu.VMEM((640,), jnp.float32),
        pltpu.VMEM((640,), jnp.float32),
        pltpu.VMEM_SHARED((16, NPAD), jnp.float32),
    ],
    mesh=_mesh,
)
def _prep(src_hbm, dst_hbm, dstr_hbm, degp_hbm, sbuf, dbuf, obuf, acc, rbuf,
          sumbuf, shacc):
    c = lax.axis_index("c")
    s = lax.axis_index("s")
    w = s * 2 + c
    zero16 = jnp.zeros((16,), jnp.float32)
    ones16 = jnp.ones((16,), jnp.float32)
    trash16 = jnp.full((16,), TRASH, jnp.int32)

    def zbody(i, _):
        acc[pl.ds(i * 16, 16)] = zero16
        return 0
    lax.fori_loop(0, NPAD // 16, zbody, 0)

    base0 = w * EW
    for j in range(EW // _CH):
        base = base0 + j * _CH
        pltpu.sync_copy(src_hbm.at[pl.ds(base, _CH)], sbuf)
        pltpu.sync_copy(dst_hbm.at[pl.ds(base, _CH)], dbuf)

        def body(i, _):
            s16 = sbuf[pl.ds(i * 16, 16)]
            d16 = dbuf[pl.ds(i * 16, 16)]
            m = s16 != d16
            obuf[pl.ds(i * 16, 16)] = jnp.where(m, d16, trash16)
            plsc.addupdate_scatter(acc, [s16], ones16, mask=m)
            return 0
        lax.fori_loop(0, _CH // 16, body, 0)
        pltpu.sync_copy(obuf, dstr_hbm.at[pl.ds(base, _CH)])

    pltpu.sync_copy(acc, shacc.at[s])
    plsc.subcore_barrier()
    col0 = s * (NPAD // 16)

    def zb2(i, _):
        sumbuf[pl.ds(i * 16, 16)] = zero16
        return 0
    lax.fori_loop(0, 640 // 16, zb2, 0)
    for r in range(16):
        pltpu.sync_copy(shacc.at[r, pl.ds(col0, 640)], rbuf)

        def ab(i, _):
            sumbuf[pl.ds(i * 16, 16)] = sumbuf[pl.ds(i * 16, 16)] + rbuf[pl.ds(i * 16, 16)]
            return 0
        lax.fori_loop(0, 640 // 16, ab, 0)
    pltpu.sync_copy(sumbuf, degp_hbm.at[c, pl.ds(col0, 640)])


_B = 200
_G = EW // _B


@functools.partial(
    pl.kernel,
    out_type=jax.ShapeDtypeStruct((2, N, D), jnp.float32),
    scratch_types=[
        pltpu.VMEM((_B,), jnp.int32),
        pltpu.VMEM((_B,), jnp.int32),
        pltpu.VMEM((_B, D), jnp.float32),
        pltpu.VMEM((64, D), jnp.float32),
        pltpu.VMEM_SHARED((NPAD, D), jnp.float32),
        pltpu.SemaphoreType.DMA,
    ],
    mesh=_mesh,
)
def _prop(y_hbm, src_hbm, dstr_hbm, zp_hbm, sidx, didx, rows, zbuf, acc, sem):
    c = lax.axis_index("c")
    s = lax.axis_index("s")
    w = s * 2 + c
    zero16 = jnp.zeros((16,), jnp.float32)

    def zb(i, _):
        zbuf[i // 8, pl.ds((i % 8) * 16, 16)] = zero16
        return 0
    lax.fori_loop(0, 64 * 8, zb, 0)
    r0 = s * (NPAD // 16)
    for q in range(10):
        pltpu.sync_copy(zbuf, acc.at[pl.ds(r0 + q * 64, 64)])
    plsc.subcore_barrier()

    base0 = w * EW

    def gbody(g, _):
        base = base0 + g * _B
        pltpu.sync_copy(src_hbm.at[pl.ds(base, _B)], sidx)
        pltpu.sync_copy(dstr_hbm.at[pl.ds(base, _B)], didx)
        pltpu.async_copy(y_hbm.at[sidx], rows, sem).wait()
        pltpu.sync_copy(rows, acc.at[didx], add=True)
        return 0
    lax.fori_loop(0, _G, gbody, 0)
    plsc.subcore_barrier()

    rt0 = s * (N // 16)
    for q in range(5):
        rr = rt0 + q * 125
        pltpu.sync_copy(acc.at[pl.ds(rr, 125)], rows.at[pl.ds(0, 125)])
        pltpu.sync_copy(rows.at[pl.ds(0, 125)], zp_hbm.at[c, pl.ds(rr, 125)])


def _k2_body(dp_ref, x_ref, dinv_ref, y1_ref):
    d = dp_ref[0] + dp_ref[1]
    good = d > 0.0
    dinv = jnp.where(good, lax.rsqrt(jnp.where(good, d, 1.0)), 0.0)
    dinv_ref[...] = dinv
    y1_ref[...] = dinv * x_ref[...]


_k2 = pl.pallas_call(
    _k2_body,
    grid=(GRID,),
    in_specs=[
        pl.BlockSpec((2, R, 1), lambda i: (0, i, 0)),
        pl.BlockSpec((R, D), lambda i: (i, 0)),
    ],
    out_specs=[
        pl.BlockSpec((R, 1), lambda i: (i, 0)),
        pl.BlockSpec((R, D), lambda i: (i, 0)),
    ],
    out_shape=[
        jax.ShapeDtypeStruct((N, 1), jnp.float32),
        jax.ShapeDtypeStruct((N, D), jnp.float32),
    ],
)


def _k4_body(zp_ref, dinv_ref, tx1_ref, y2_ref):
    z = zp_ref[0] + zp_ref[1]
    dinv = dinv_ref[...]
    tx1 = -dinv * z
    tx1_ref[...] = tx1
    y2_ref[...] = dinv * tx1


_k4 = pl.pallas_call(
    _k4_body,
    grid=(GRID,),
    in_specs=[
        pl.BlockSpec((2, R, D), lambda i: (0, i, 0)),
        pl.BlockSpec((R, 1), lambda i: (i, 0)),
    ],
    out_specs=[
        pl.BlockSpec((R, D), lambda i: (i, 0)),
        pl.BlockSpec((R, D), lambda i: (i, 0)),
    ],
    out_shape=[
        jax.ShapeDtypeStruct((N, D), jnp.float32),
        jax.ShapeDtypeStruct((N, D), jnp.float32),
    ],
)


def _k6_body(x_ref, tx1_ref, zp_ref, dinv_ref, w_ref, b_ref, out_ref):
    z = zp_ref[0] + zp_ref[1]
    x = x_ref[...]
    tx1 = tx1_ref[...]
    tx2 = -2.0 * dinv_ref[...] * z - x
    out = jnp.dot(x, w_ref[0], preferred_element_type=jnp.float32)
    out = out + jnp.dot(tx1, w_ref[1], preferred_element_type=jnp.float32)
    out = out + jnp.dot(tx2, w_ref[2], preferred_element_type=jnp.float32)
    out_ref[...] = out + b_ref[...]


_k6 = pl.pallas_call(
    _k6_body,
    grid=(GRID,),
    in_specs=[
        pl.BlockSpec((R, D), lambda i: (i, 0)),
        pl.BlockSpec((R, D), lambda i: (i, 0)),
        pl.BlockSpec((2, R, D), lambda i: (0, i, 0)),
        pl.BlockSpec((R, 1), lambda i: (i, 0)),
        pl.BlockSpec((3, D, D), lambda i: (0, 0, 0)),
        pl.BlockSpec((1, D), lambda i: (0, 0)),
    ],
    out_specs=pl.BlockSpec((R, D), lambda i: (i, 0)),
    out_shape=jax.ShapeDtypeStruct((N, D), jnp.float32),
)


def kernel(x, edge_index, weight, bias):
    x = x.astype(jnp.float32)
    src = edge_index[0]
    dst = edge_index[1]
    dstr, degp = _prep(src, dst)
    dp = degp[:, :N, None]
    dinv, y1 = _k2(dp, x)
    zp1 = _prop(y1, src, dstr)
    tx1, y2 = _k4(zp1, dinv)
    zp2 = _prop(y2, src, dstr)
    out = _k6(x, tx1, zp2, dinv, weight, bias.reshape(1, D))
    ixz = jnp.zeros((N,), jnp.float32)
    skl = jnp.zeros((), jnp.float32)
    return out, ixz, skl


# speedup vs baseline: 15.1459x; 15.1459x over previous
import functools

import jax
import jax.numpy as jnp
from jax import lax
from jax.experimental import pallas as pl
from jax.experimental.pallas import tpu as pltpu, tpu_sc as plsc

N = 10000
E = 320000
D = 128
NPAD = 10240
TRASH = 10000
NW = 32
EW = E // NW
R = 1000
GRID = N // R

_mesh = plsc.VectorSubcoreMesh(core_axis_name="c", subcore_axis_name="s")

_CH = 2000


@functools.partial(
    pl.kernel,
    out_type=[
        jax.ShapeDtypeStruct((E,), jnp.int32),
        jax.ShapeDtypeStruct((2, NPAD), jnp.float32),
    ],
    scratch_types=[
        pltpu.VMEM((_CH,), jnp.int32),
        pltpu.VMEM((_CH,), jnp.int32),
        pltpu.VMEM((_CH,), jnp.int32),
        pltpu.VMEM((NPAD,), jnp.float32),
        pltpu.VMEM((640,), jnp.float32),
        pltpu.VMEM((640,), jnp.float32),
        pltpu.VMEM_SHARED((16, NPAD), jnp.float32),
    ],
    mesh=_mesh,
    compiler_params=pltpu.CompilerParams(needs_layout_passes=False),
)
def _prep(src_hbm, dst_hbm, dstr_hbm, degp_hbm, sbuf, dbuf, obuf, acc, rbuf,
          sumbuf, shacc):
    c = lax.axis_index("c")
    s = lax.axis_index("s")
    w = s * 2 + c
    zero16 = jnp.zeros((16,), jnp.float32)
    ones16 = jnp.ones((16,), jnp.float32)
    trash16 = jnp.full((16,), TRASH, jnp.int32)

    def zbody(i, _):
        acc[pl.ds(i * 16, 16)] = zero16
        return 0
    lax.fori_loop(0, NPAD // 16, zbody, 0)

    base0 = w * EW
    for j in range(EW // _CH):
        base = base0 + j * _CH
        pltpu.sync_copy(src_hbm.at[pl.ds(base, _CH)], sbuf)
        pltpu.sync_copy(dst_hbm.at[pl.ds(base, _CH)], dbuf)

        def body(i, _):
            s16 = sbuf[pl.ds(i * 16, 16)]
            d16 = dbuf[pl.ds(i * 16, 16)]
            m = s16 != d16
            obuf[pl.ds(i * 16, 16)] = jnp.where(m, d16, trash16)
            plsc.addupdate_scatter(acc, [s16], ones16, mask=m)
            return 0
        lax.fori_loop(0, _CH // 16, body, 0)
        pltpu.sync_copy(obuf, dstr_hbm.at[pl.ds(base, _CH)])

    pltpu.sync_copy(acc, shacc.at[s])
    plsc.subcore_barrier()
    col0 = s * (NPAD // 16)

    def zb2(i, _):
        sumbuf[pl.ds(i * 16, 16)] = zero16
        return 0
    lax.fori_loop(0, 640 // 16, zb2, 0)
    for r in range(16):
        pltpu.sync_copy(shacc.at[r, pl.ds(col0, 640)], rbuf)

        def ab(i, _):
            sumbuf[pl.ds(i * 16, 16)] = sumbuf[pl.ds(i * 16, 16)] + rbuf[pl.ds(i * 16, 16)]
            return 0
        lax.fori_loop(0, 640 // 16, ab, 0)
    pltpu.sync_copy(sumbuf, degp_hbm.at[c, pl.ds(col0, 640)])


_B = 200
_G = EW // _B


@functools.partial(
    pl.kernel,
    out_type=jax.ShapeDtypeStruct((2, NPAD, D), jnp.float32),
    scratch_types=[
        pltpu.VMEM((_B,), jnp.int32),
        pltpu.VMEM((_B,), jnp.int32),
        pltpu.VMEM((_B, D), jnp.float32),
        pltpu.VMEM((64, D), jnp.float32),
        pltpu.VMEM_SHARED((NPAD, D), jnp.float32),
        pltpu.SemaphoreType.DMA,
    ],
    mesh=_mesh,
    compiler_params=pltpu.CompilerParams(needs_layout_passes=False),
)
def _prop(y_hbm, src_hbm, dstr_hbm, zp_hbm, sidx, didx, rows, zbuf, acc, sem):
    c = lax.axis_index("c")
    s = lax.axis_index("s")
    w = s * 2 + c
    zero16 = jnp.zeros((16,), jnp.float32)

    def zb(i, _):
        zbuf[i // 8, pl.ds((i % 8) * 16, 16)] = zero16
        return 0
    lax.fori_loop(0, 64 * 8, zb, 0)
    r0 = s * (NPAD // 16)
    for q in range(10):
        pltpu.sync_copy(zbuf, acc.at[pl.ds(r0 + q * 64, 64)])
    plsc.subcore_barrier()

    base0 = w * EW

    def gbody(g, _):
        base = base0 + g * _B
        pltpu.sync_copy(src_hbm.at[pl.ds(base, _B)], sidx)
        pltpu.sync_copy(dstr_hbm.at[pl.ds(base, _B)], didx)
        pltpu.async_copy(y_hbm.at[sidx], rows, sem).wait()
        pltpu.sync_copy(rows, acc.at[didx], add=True)
        return 0
    lax.fori_loop(0, _G, gbody, 0)
    plsc.subcore_barrier()

    rt0 = s * (NPAD // 16)
    for q in range(10):
        rr = rt0 + q * 64
        pltpu.sync_copy(acc.at[pl.ds(rr, 64)], rows.at[pl.ds(0, 64)])
        pltpu.sync_copy(rows.at[pl.ds(0, 64)], zp_hbm.at[c, pl.ds(rr, 64)])


def _k2_body(dp_ref, x_ref, dinv_ref, y1_ref):
    d = dp_ref[0] + dp_ref[1]
    good = d > 0.0
    dinv = jnp.where(good, lax.rsqrt(jnp.where(good, d, 1.0)), 0.0)
    dinv_ref[...] = dinv
    y1_ref[...] = dinv * x_ref[...]


_k2 = pl.pallas_call(
    _k2_body,
    grid=(GRID,),
    in_specs=[
        pl.BlockSpec((2, R, 1), lambda i: (0, i, 0)),
        pl.BlockSpec((R, D), lambda i: (i, 0)),
    ],
    out_specs=[
        pl.BlockSpec((R, 1), lambda i: (i, 0)),
        pl.BlockSpec((R, D), lambda i: (i, 0)),
    ],
    out_shape=[
        jax.ShapeDtypeStruct((N, 1), jnp.float32),
        jax.ShapeDtypeStruct((N, D), jnp.float32),
    ],
)


def _k4_body(zp_ref, dinv_ref, tx1_ref, y2_ref):
    z = zp_ref[0] + zp_ref[1]
    dinv = dinv_ref[...]
    tx1 = -dinv * z
    tx1_ref[...] = tx1
    y2_ref[...] = dinv * tx1


_k4 = pl.pallas_call(
    _k4_body,
    grid=(GRID,),
    in_specs=[
        pl.BlockSpec((2, R, D), lambda i: (0, i, 0)),
        pl.BlockSpec((R, 1), lambda i: (i, 0)),
    ],
    out_specs=[
        pl.BlockSpec((R, D), lambda i: (i, 0)),
        pl.BlockSpec((R, D), lambda i: (i, 0)),
    ],
    out_shape=[
        jax.ShapeDtypeStruct((N, D), jnp.float32),
        jax.ShapeDtypeStruct((N, D), jnp.float32),
    ],
)


def _k6_body(x_ref, tx1_ref, zp_ref, dinv_ref, w_ref, b_ref, out_ref):
    z = zp_ref[0] + zp_ref[1]
    x = x_ref[...]
    tx1 = tx1_ref[...]
    tx2 = -2.0 * dinv_ref[...] * z - x
    out = jnp.dot(x, w_ref[0], preferred_element_type=jnp.float32)
    out = out + jnp.dot(tx1, w_ref[1], preferred_element_type=jnp.float32)
    out = out + jnp.dot(tx2, w_ref[2], preferred_element_type=jnp.float32)
    out_ref[...] = out + b_ref[...]


_k6 = pl.pallas_call(
    _k6_body,
    grid=(GRID,),
    in_specs=[
        pl.BlockSpec((R, D), lambda i: (i, 0)),
        pl.BlockSpec((R, D), lambda i: (i, 0)),
        pl.BlockSpec((2, R, D), lambda i: (0, i, 0)),
        pl.BlockSpec((R, 1), lambda i: (i, 0)),
        pl.BlockSpec((3, D, D), lambda i: (0, 0, 0)),
        pl.BlockSpec((1, D), lambda i: (0, 0)),
    ],
    out_specs=pl.BlockSpec((R, D), lambda i: (i, 0)),
    out_shape=jax.ShapeDtypeStruct((N, D), jnp.float32),
)


def kernel(x, edge_index, weight, bias):
    x = x.astype(jnp.float32)
    src = edge_index[0]
    dst = edge_index[1]
    dstr, degp = _prep(src, dst)
    dp = degp[:, :, None]
    dinv, y1 = _k2(dp, x)
    zp1 = _prop(y1, src, dstr)
    tx1, y2 = _k4(zp1, dinv)
    zp2 = _prop(y2, src, dstr)
    out = _k6(x, tx1, zp2, dinv, weight, bias.reshape(1, D))
    ixz = jnp.zeros((N,), jnp.float32)
    skl = jnp.zeros((), jnp.float32)
    return out, ixz, skl


# speedup vs baseline: 19.5630x; 1.2916x over previous
import functools

import jax
import jax.numpy as jnp
from jax import lax
from jax.experimental import pallas as pl
from jax.experimental.pallas import tpu as pltpu, tpu_sc as plsc

N = 10000
E = 320000
D = 128
NPAD = 10240
TRASH = 10000
NW = 32
EW = E // NW
R = 1000
GRID = N // R

_mesh = plsc.VectorSubcoreMesh(core_axis_name="c", subcore_axis_name="s")

_CH = 2000


@functools.partial(
    pl.kernel,
    out_type=[
        jax.ShapeDtypeStruct((E,), jnp.int32),
        jax.ShapeDtypeStruct((2, NPAD), jnp.float32),
    ],
    scratch_types=[
        pltpu.VMEM((_CH,), jnp.int32),
        pltpu.VMEM((_CH,), jnp.int32),
        pltpu.VMEM((_CH,), jnp.int32),
        pltpu.VMEM((NPAD,), jnp.float32),
        pltpu.VMEM((640,), jnp.float32),
        pltpu.VMEM((640,), jnp.float32),
        pltpu.VMEM_SHARED((16, NPAD), jnp.float32),
    ],
    mesh=_mesh,
    compiler_params=pltpu.CompilerParams(needs_layout_passes=False),
)
def _prep(src_hbm, dst_hbm, dstr_hbm, degp_hbm, sbuf, dbuf, obuf, acc, rbuf,
          sumbuf, shacc):
    c = lax.axis_index("c")
    s = lax.axis_index("s")
    w = s * 2 + c
    zero16 = jnp.zeros((16,), jnp.float32)
    ones16 = jnp.ones((16,), jnp.float32)
    trash16 = jnp.full((16,), TRASH, jnp.int32)

    def zbody(i, _):
        acc[pl.ds(i * 16, 16)] = zero16
        return 0
    lax.fori_loop(0, NPAD // 16, zbody, 0)

    base0 = w * EW
    for j in range(EW // _CH):
        base = base0 + j * _CH
        pltpu.sync_copy(src_hbm.at[pl.ds(base, _CH)], sbuf)
        pltpu.sync_copy(dst_hbm.at[pl.ds(base, _CH)], dbuf)

        def body(i, _):
            s16 = sbuf[pl.ds(i * 16, 16)]
            d16 = dbuf[pl.ds(i * 16, 16)]
            m = s16 != d16
            obuf[pl.ds(i * 16, 16)] = jnp.where(m, d16, trash16)
            plsc.addupdate_scatter(acc, [s16], ones16, mask=m)
            return 0
        lax.fori_loop(0, _CH // 16, body, 0)
        pltpu.sync_copy(obuf, dstr_hbm.at[pl.ds(base, _CH)])

    pltpu.sync_copy(acc, shacc.at[s])
    plsc.subcore_barrier()
    col0 = s * (NPAD // 16)

    def zb2(i, _):
        sumbuf[pl.ds(i * 16, 16)] = zero16
        return 0
    lax.fori_loop(0, 640 // 16, zb2, 0)
    for r in range(16):
        pltpu.sync_copy(shacc.at[r, pl.ds(col0, 640)], rbuf)

        def ab(i, _):
            sumbuf[pl.ds(i * 16, 16)] = sumbuf[pl.ds(i * 16, 16)] + rbuf[pl.ds(i * 16, 16)]
            return 0
        lax.fori_loop(0, 640 // 16, ab, 0)
    pltpu.sync_copy(sumbuf, degp_hbm.at[c, pl.ds(col0, 640)])


_B = 100
_G = EW // _B


@functools.partial(
    pl.kernel,
    out_type=jax.ShapeDtypeStruct((2, NPAD, D), jnp.float32),
    scratch_types=[
        pltpu.VMEM((_B,), jnp.int32),
        pltpu.VMEM((_B,), jnp.int32),
        pltpu.VMEM((_B,), jnp.int32),
        pltpu.VMEM((_B,), jnp.int32),
        pltpu.VMEM((_B, D), jnp.float32),
        pltpu.VMEM((_B, D), jnp.float32),
        pltpu.VMEM_SHARED((NPAD, D), jnp.float32),
        pltpu.SemaphoreType.DMA,
        pltpu.SemaphoreType.DMA,
        pltpu.SemaphoreType.DMA,
        pltpu.SemaphoreType.DMA,
    ],
    mesh=_mesh,
    compiler_params=pltpu.CompilerParams(needs_layout_passes=False),
)
def _prop(y_hbm, src_hbm, dstr_hbm, zp_hbm, sidx0, sidx1, didx0, didx1,
          rows0, rows1, acc, gsem0, gsem1, isem0, isem1):
    c = lax.axis_index("c")
    s = lax.axis_index("s")
    w = s * 2 + c
    zero16 = jnp.zeros((16,), jnp.float32)

    def zb(i, _):
        rows0[i // 8, pl.ds((i % 8) * 16, 16)] = zero16
        return 0
    lax.fori_loop(0, 64 * 8, zb, 0)
    r0 = s * (NPAD // 16)
    for q in range(10):
        pltpu.sync_copy(rows0.at[pl.ds(0, 64)], acc.at[pl.ds(r0 + q * 64, 64)])
    plsc.subcore_barrier()

    pltpu.sync_copy(src_hbm.at[w, 0], sidx0)
    pltpu.sync_copy(dstr_hbm.at[w, 0], didx0)
    pltpu.async_copy(y_hbm.at[sidx0], rows0, gsem0)
    pltpu.async_copy(src_hbm.at[w, 1], sidx1, isem1)
    pltpu.async_copy(dstr_hbm.at[w, 1], didx1, isem1)

    def gbody(i, _):
        g0 = 2 * i
        g1 = g0 + 1
        pltpu.make_async_copy(y_hbm.at[sidx0], rows0, gsem0).wait()
        pltpu.make_async_copy(src_hbm.at[w, g1], sidx1, isem1).wait()
        pltpu.make_async_copy(dstr_hbm.at[w, g1], didx1, isem1).wait()
        pltpu.async_copy(y_hbm.at[sidx1], rows1, gsem1)
        pltpu.sync_copy(rows0, acc.at[didx0], add=True)

        @pl.when(g0 + 2 < _G)
        def _pf0():
            pltpu.async_copy(src_hbm.at[w, g0 + 2], sidx0, isem0)
            pltpu.async_copy(dstr_hbm.at[w, g0 + 2], didx0, isem0)

        pltpu.make_async_copy(y_hbm.at[sidx1], rows1, gsem1).wait()

        @pl.when(g0 + 2 < _G)
        def _g2():
            pltpu.make_async_copy(src_hbm.at[w, g0 + 2], sidx0, isem0).wait()
            pltpu.make_async_copy(dstr_hbm.at[w, g0 + 2], didx0, isem0).wait()
            pltpu.async_copy(y_hbm.at[sidx0], rows0, gsem0)

        pltpu.sync_copy(rows1, acc.at[didx1], add=True)

        @pl.when(g0 + 3 < _G)
        def _pf1():
            pltpu.async_copy(src_hbm.at[w, g0 + 3], sidx1, isem1)
            pltpu.async_copy(dstr_hbm.at[w, g0 + 3], didx1, isem1)
        return 0
    lax.fori_loop(0, _G // 2, gbody, 0)
    plsc.subcore_barrier()

    rt0 = s * (NPAD // 16)
    for q in range(10):
        rr = rt0 + q * 64
        pltpu.sync_copy(acc.at[pl.ds(rr, 64)], rows0.at[pl.ds(0, 64)])
        pltpu.sync_copy(rows0.at[pl.ds(0, 64)], zp_hbm.at[c, pl.ds(rr, 64)])


def _k2_body(dp_ref, x_ref, dinv_ref, y1_ref):
    d = dp_ref[0] + dp_ref[1]
    good = d > 0.0
    dinv = jnp.where(good, lax.rsqrt(jnp.where(good, d, 1.0)), 0.0)
    dinv_ref[...] = dinv
    y1_ref[...] = dinv * x_ref[...]


_k2 = pl.pallas_call(
    _k2_body,
    grid=(GRID,),
    in_specs=[
        pl.BlockSpec((2, R, 1), lambda i: (0, i, 0)),
        pl.BlockSpec((R, D), lambda i: (i, 0)),
    ],
    out_specs=[
        pl.BlockSpec((R, 1), lambda i: (i, 0)),
        pl.BlockSpec((R, D), lambda i: (i, 0)),
    ],
    out_shape=[
        jax.ShapeDtypeStruct((N, 1), jnp.float32),
        jax.ShapeDtypeStruct((N, D), jnp.float32),
    ],
)


def _k4_body(zp_ref, dinv_ref, tx1_ref, y2_ref):
    z = zp_ref[0] + zp_ref[1]
    dinv = dinv_ref[...]
    tx1 = -dinv * z
    tx1_ref[...] = tx1
    y2_ref[...] = dinv * tx1


_k4 = pl.pallas_call(
    _k4_body,
    grid=(GRID,),
    in_specs=[
        pl.BlockSpec((2, R, D), lambda i: (0, i, 0)),
        pl.BlockSpec((R, 1), lambda i: (i, 0)),
    ],
    out_specs=[
        pl.BlockSpec((R, D), lambda i: (i, 0)),
        pl.BlockSpec((R, D), lambda i: (i, 0)),
    ],
    out_shape=[
        jax.ShapeDtypeStruct((N, D), jnp.float32),
        jax.ShapeDtypeStruct((N, D), jnp.float32),
    ],
)


def _k6_body(x_ref, tx1_ref, zp_ref, dinv_ref, w_ref, b_ref, out_ref):
    z = zp_ref[0] + zp_ref[1]
    x = x_ref[...]
    tx1 = tx1_ref[...]
    tx2 = -2.0 * dinv_ref[...] * z - x
    out = jnp.dot(x, w_ref[0], preferred_element_type=jnp.float32)
    out = out + jnp.dot(tx1, w_ref[1], preferred_element_type=jnp.float32)
    out = out + jnp.dot(tx2, w_ref[2], preferred_element_type=jnp.float32)
    out_ref[...] = out + b_ref[...]


_k6 = pl.pallas_call(
    _k6_body,
    grid=(GRID,),
    in_specs=[
        pl.BlockSpec((R, D), lambda i: (i, 0)),
        pl.BlockSpec((R, D), lambda i: (i, 0)),
        pl.BlockSpec((2, R, D), lambda i: (0, i, 0)),
        pl.BlockSpec((R, 1), lambda i: (i, 0)),
        pl.BlockSpec((3, D, D), lambda i: (0, 0, 0)),
        pl.BlockSpec((1, D), lambda i: (0, 0)),
    ],
    out_specs=pl.BlockSpec((R, D), lambda i: (i, 0)),
    out_shape=jax.ShapeDtypeStruct((N, D), jnp.float32),
)


def kernel(x, edge_index, weight, bias):
    x = x.astype(jnp.float32)
    src = edge_index[0]
    dst = edge_index[1]
    dstr, degp = _prep(src, dst)
    src3 = src.reshape(NW, _G, _B)
    dst3 = dstr.reshape(NW, _G, _B)
    dp = degp[:, :, None]
    dinv, y1 = _k2(dp, x)
    zp1 = _prop(y1, src3, dst3)
    tx1, y2 = _k4(zp1, dinv)
    zp2 = _prop(y2, src3, dst3)
    out = _k6(x, tx1, zp2, dinv, weight, bias.reshape(1, D))
    ixz = jnp.zeros((N,), jnp.float32)
    skl = jnp.zeros((), jnp.float32)
    return out, ixz, skl


# speedup vs baseline: 20.7916x; 1.0628x over previous
import functools

import jax
import jax.numpy as jnp
from jax import lax
from jax.experimental import pallas as pl
from jax.experimental.pallas import tpu as pltpu, tpu_sc as plsc

N = 10000
E = 320000
D = 128
NPAD = 10240
TRASH = 10000
NW = 32
EW = E // NW
R = 1000
GRID = N // R

_mesh = plsc.VectorSubcoreMesh(core_axis_name="c", subcore_axis_name="s")

_CH = 2000


@functools.partial(
    pl.kernel,
    out_type=[
        jax.ShapeDtypeStruct((E,), jnp.int32),
        jax.ShapeDtypeStruct((2, NPAD), jnp.float32),
    ],
    scratch_types=[
        pltpu.VMEM((_CH,), jnp.int32),
        pltpu.VMEM((_CH,), jnp.int32),
        pltpu.VMEM((_CH,), jnp.int32),
        pltpu.VMEM((NPAD,), jnp.float32),
        pltpu.VMEM((640,), jnp.float32),
        pltpu.VMEM((640,), jnp.float32),
        pltpu.VMEM_SHARED((16, NPAD), jnp.float32),
    ],
    mesh=_mesh,
    compiler_params=pltpu.CompilerParams(needs_layout_passes=False),
)
def _prep(src_hbm, dst_hbm, dstr_hbm, degp_hbm, sbuf, dbuf, obuf, acc, rbuf,
          sumbuf, shacc):
    c = lax.axis_index("c")
    s = lax.axis_index("s")
    w = s * 2 + c
    zero16 = jnp.zeros((16,), jnp.float32)
    ones16 = jnp.ones((16,), jnp.float32)
    trash16 = jnp.full((16,), TRASH, jnp.int32)

    def zbody(i, _):
        acc[pl.ds(i * 16, 16)] = zero16
        return 0
    lax.fori_loop(0, NPAD // 16, zbody, 0)

    base0 = w * EW
    for j in range(EW // _CH):
        base = base0 + j * _CH
        pltpu.sync_copy(src_hbm.at[pl.ds(base, _CH)], sbuf)
        pltpu.sync_copy(dst_hbm.at[pl.ds(base, _CH)], dbuf)

        def body(i, _):
            s16 = sbuf[pl.ds(i * 16, 16)]
            d16 = dbuf[pl.ds(i * 16, 16)]
            m = s16 != d16
            obuf[pl.ds(i * 16, 16)] = jnp.where(m, d16, trash16)
            plsc.addupdate_scatter(acc, [s16], ones16, mask=m)
            return 0
        lax.fori_loop(0, _CH // 16, body, 0)
        pltpu.sync_copy(obuf, dstr_hbm.at[pl.ds(base, _CH)])

    pltpu.sync_copy(acc, shacc.at[s])
    plsc.subcore_barrier()
    col0 = s * (NPAD // 16)

    def zb2(i, _):
        sumbuf[pl.ds(i * 16, 16)] = zero16
        return 0
    lax.fori_loop(0, 640 // 16, zb2, 0)
    for r in range(16):
        pltpu.sync_copy(shacc.at[r, pl.ds(col0, 640)], rbuf)

        def ab(i, _):
            sumbuf[pl.ds(i * 16, 16)] = sumbuf[pl.ds(i * 16, 16)] + rbuf[pl.ds(i * 16, 16)]
            return 0
        lax.fori_loop(0, 640 // 16, ab, 0)
    pltpu.sync_copy(sumbuf, degp_hbm.at[c, pl.ds(col0, 640)])


_B = 125
_G = EW // _B


@functools.partial(
    pl.kernel,
    out_type=jax.ShapeDtypeStruct((2, NPAD, D), jnp.float32),
    scratch_types=[
        pltpu.VMEM((_B,), jnp.int32),
        pltpu.VMEM((_B,), jnp.int32),
        pltpu.VMEM((_B,), jnp.int32),
        pltpu.VMEM((_B,), jnp.int32),
        pltpu.VMEM((_B, D), jnp.float32),
        pltpu.VMEM((_B, D), jnp.float32),
        pltpu.VMEM_SHARED((NPAD, D), jnp.float32),
        pltpu.SemaphoreType.DMA,
        pltpu.SemaphoreType.DMA,
        pltpu.SemaphoreType.DMA,
        pltpu.SemaphoreType.DMA,
    ],
    mesh=_mesh,
    compiler_params=pltpu.CompilerParams(needs_layout_passes=False),
)
def _prop(y_hbm, src_hbm, dstr_hbm, zp_hbm, sidx0, sidx1, didx0, didx1,
          rows0, rows1, acc, gsem0, gsem1, isem0, isem1):
    c = lax.axis_index("c")
    s = lax.axis_index("s")
    w = s * 2 + c
    zero16 = jnp.zeros((16,), jnp.float32)

    def zb(i, _):
        rows0[i // 8, pl.ds((i % 8) * 16, 16)] = zero16
        return 0
    lax.fori_loop(0, 64 * 8, zb, 0)
    r0 = s * (NPAD // 16)
    for q in range(10):
        pltpu.sync_copy(rows0.at[pl.ds(0, 64)], acc.at[pl.ds(r0 + q * 64, 64)])
    plsc.subcore_barrier()

    pltpu.sync_copy(src_hbm.at[w, 0], sidx0)
    pltpu.sync_copy(dstr_hbm.at[w, 0], didx0)
    pltpu.async_copy(y_hbm.at[sidx0], rows0, gsem0)
    pltpu.async_copy(src_hbm.at[w, 1], sidx1, isem1)
    pltpu.async_copy(dstr_hbm.at[w, 1], didx1, isem1)

    def gbody(i, _):
        g0 = 2 * i
        g1 = g0 + 1
        pltpu.make_async_copy(y_hbm.at[sidx0], rows0, gsem0).wait()
        pltpu.make_async_copy(src_hbm.at[w, g1], sidx1, isem1).wait()
        pltpu.make_async_copy(dstr_hbm.at[w, g1], didx1, isem1).wait()
        pltpu.async_copy(y_hbm.at[sidx1], rows1, gsem1)
        pltpu.sync_copy(rows0, acc.at[didx0], add=True)

        @pl.when(g0 + 2 < _G)
        def _pf0():
            pltpu.async_copy(src_hbm.at[w, g0 + 2], sidx0, isem0)
            pltpu.async_copy(dstr_hbm.at[w, g0 + 2], didx0, isem0)

        pltpu.make_async_copy(y_hbm.at[sidx1], rows1, gsem1).wait()

        @pl.when(g0 + 2 < _G)
        def _g2():
            pltpu.make_async_copy(src_hbm.at[w, g0 + 2], sidx0, isem0).wait()
            pltpu.make_async_copy(dstr_hbm.at[w, g0 + 2], didx0, isem0).wait()
            pltpu.async_copy(y_hbm.at[sidx0], rows0, gsem0)

        pltpu.sync_copy(rows1, acc.at[didx1], add=True)

        @pl.when(g0 + 3 < _G)
        def _pf1():
            pltpu.async_copy(src_hbm.at[w, g0 + 3], sidx1, isem1)
            pltpu.async_copy(dstr_hbm.at[w, g0 + 3], didx1, isem1)
        return 0
    lax.fori_loop(0, _G // 2, gbody, 0)
    plsc.subcore_barrier()

    rt0 = s * (NPAD // 16)
    for q in range(10):
        rr = rt0 + q * 64
        pltpu.sync_copy(acc.at[pl.ds(rr, 64)], rows0.at[pl.ds(0, 64)])
        pltpu.sync_copy(rows0.at[pl.ds(0, 64)], zp_hbm.at[c, pl.ds(rr, 64)])


def _k2_body(dp_ref, x_ref, dinv_ref, y1_ref):
    d = dp_ref[0] + dp_ref[1]
    good = d > 0.0
    dinv = jnp.where(good, lax.rsqrt(jnp.where(good, d, 1.0)), 0.0)
    dinv_ref[...] = dinv
    y1_ref[...] = dinv * x_ref[...]


_k2 = pl.pallas_call(
    _k2_body,
    grid=(GRID,),
    in_specs=[
        pl.BlockSpec((2, R, 1), lambda i: (0, i, 0)),
        pl.BlockSpec((R, D), lambda i: (i, 0)),
    ],
    out_specs=[
        pl.BlockSpec((R, 1), lambda i: (i, 0)),
        pl.BlockSpec((R, D), lambda i: (i, 0)),
    ],
    out_shape=[
        jax.ShapeDtypeStruct((N, 1), jnp.float32),
        jax.ShapeDtypeStruct((N, D), jnp.float32),
    ],
)


def _k4_body(zp_ref, dinv_ref, tx1_ref, y2_ref):
    z = zp_ref[0] + zp_ref[1]
    dinv = dinv_ref[...]
    tx1 = -dinv * z
    tx1_ref[...] = tx1
    y2_ref[...] = dinv * tx1


_k4 = pl.pallas_call(
    _k4_body,
    grid=(GRID,),
    in_specs=[
        pl.BlockSpec((2, R, D), lambda i: (0, i, 0)),
        pl.BlockSpec((R, 1), lambda i: (i, 0)),
    ],
    out_specs=[
        pl.BlockSpec((R, D), lambda i: (i, 0)),
        pl.BlockSpec((R, D), lambda i: (i, 0)),
    ],
    out_shape=[
        jax.ShapeDtypeStruct((N, D), jnp.float32),
        jax.ShapeDtypeStruct((N, D), jnp.float32),
    ],
)


def _k6_body(x_ref, tx1_ref, zp_ref, dinv_ref, w_ref, b_ref, out_ref):
    z = zp_ref[0] + zp_ref[1]
    x = x_ref[...]
    tx1 = tx1_ref[...]
    tx2 = -2.0 * dinv_ref[...] * z - x
    out = jnp.dot(x, w_ref[0], preferred_element_type=jnp.float32)
    out = out + jnp.dot(tx1, w_ref[1], preferred_element_type=jnp.float32)
    out = out + jnp.dot(tx2, w_ref[2], preferred_element_type=jnp.float32)
    out_ref[...] = out + b_ref[...]


_k6 = pl.pallas_call(
    _k6_body,
    grid=(GRID,),
    in_specs=[
        pl.BlockSpec((R, D), lambda i: (i, 0)),
        pl.BlockSpec((R, D), lambda i: (i, 0)),
        pl.BlockSpec((2, R, D), lambda i: (0, i, 0)),
        pl.BlockSpec((R, 1), lambda i: (i, 0)),
        pl.BlockSpec((3, D, D), lambda i: (0, 0, 0)),
        pl.BlockSpec((1, D), lambda i: (0, 0)),
    ],
    out_specs=pl.BlockSpec((R, D), lambda i: (i, 0)),
    out_shape=jax.ShapeDtypeStruct((N, D), jnp.float32),
)


def kernel(x, edge_index, weight, bias):
    x = x.astype(jnp.float32)
    src = edge_index[0]
    dst = edge_index[1]
    dstr, degp = _prep(src, dst)
    src3 = src.reshape(NW, _G, _B)
    dst3 = dstr.reshape(NW, _G, _B)
    dp = degp[:, :, None]
    dinv, y1 = _k2(dp, x)
    zp1 = _prop(y1, src3, dst3)
    tx1, y2 = _k4(zp1, dinv)
    zp2 = _prop(y2, src3, dst3)
    out = _k6(x, tx1, zp2, dinv, weight, bias.reshape(1, D))
    ixz = jnp.zeros((N,), jnp.float32)
    skl = jnp.zeros((), jnp.float32)
    return out, ixz, skl
